# baseline (speedup 1.0000x reference)
import functools

import jax
import jax.numpy as jnp
from jax import lax
from jax.experimental import pallas as pl
from jax.experimental.pallas import tpu as pltpu
from jax.experimental.pallas import tpu_sc as plsc

_K = 512
_N = 2 * 576 * 96
_NC = 2
_NS = 16
_NW = _NC * _NS
_PER = _N // _NW
_L = 16
_EPS = _K // _NS


def _body(h_hbm, eb_hbm, out_hbm, x_v, o_v, eb_v, sv_v, mi_v,
          st_val, st_rank, st_macc, all_val, all_rank, all_macc,
          spm_val, spm_rank, spm_macc):
    s = lax.axis_index("s")
    c = lax.axis_index("c")
    wid = s * _NC + c
    base = wid * _PER
    pltpu.sync_copy(eb_hbm, eb_v)
    pltpu.sync_copy(h_hbm.at[pl.ds(base, _PER)], x_v)

    ebase = s * _EPS
    mine = [eb_v[pl.ds(ebase + v * _L, _L)] for v in range(_EPS // _L)]
    myidx = [lax.iota(jnp.int32, _L) + (ebase + v * _L)
             for v in range(_EPS // _L)]
    zeros = jnp.zeros((_L,), jnp.int32)
    init = tuple([zeros] * (_EPS // _L) + [zeros - 1] * (_EPS // _L))

    @plsc.parallel_loop(0, _K, unroll=16, carry=init)
    def rank_loop(k, carry):
        cnts = list(carry[: _EPS // _L])
        maccs = list(carry[_EPS // _L:])
        kb = jnp.full((_L,), k, jnp.int32)
        ev = plsc.load_gather(eb_v, [kb])
        for v in range(_EPS // _L):
            eq = ev == mine[v]
            before = (ev < mine[v]) | (eq & (kb < myidx[v]))
            cnts[v] = cnts[v] + before.astype(jnp.int32)
            maccs[v] = jnp.maximum(maccs[v], jnp.where(eq, kb, -1))
        return tuple(cnts + maccs)

    for v in range(_EPS // _L):
        st_val[pl.ds(v * _L, _L)] = mine[v]
        st_rank[pl.ds(v * _L, _L)] = rank_loop[v]
        st_macc[pl.ds(v * _L, _L)] = rank_loop[_EPS // _L + v].astype(jnp.float32)
    pltpu.sync_copy(st_val, spm_val.at[pl.ds(ebase, _EPS)])
    pltpu.sync_copy(st_rank, spm_rank.at[pl.ds(ebase, _EPS)])
    pltpu.sync_copy(st_macc, spm_macc.at[pl.ds(ebase, _EPS)])
    plsc.subcore_barrier()
    pltpu.sync_copy(spm_val, all_val)
    pltpu.sync_copy(spm_rank, all_rank)
    pltpu.sync_copy(spm_macc, all_macc)
    for b in range(_K // _L):
        r = all_rank[pl.ds(b * _L, _L)]
        plsc.store_scatter(sv_v, [r], all_val[pl.ds(b * _L, _L)])
        plsc.store_scatter(mi_v, [r], all_macc[pl.ds(b * _L, _L)])

    @plsc.parallel_loop(0, _PER // _L, unroll=12)
    def body(i):
        x = x_v[pl.ds(i * _L, _L)]
        j = jnp.zeros((_L,), jnp.int32)
        step = _K // 2
        while step >= 1:
            probe = j + (step - 1)
            v = plsc.load_gather(sv_v, [probe])
            j = jnp.where(v < x, j + step, j)
            step //= 2
        lo = jnp.maximum(j - 1, 0)
        vlo = plsc.load_gather(sv_v, [lo])
        vhi = plsc.load_gather(sv_v, [j])
        milo = plsc.load_gather(mi_v, [lo])
        mihi = plsc.load_gather(mi_v, [j])
        dlo = jnp.abs(x - vlo)
        dhi = jnp.abs(vhi - x)
        pick_hi = (dhi < dlo) | ((dhi == dlo) & (mihi > milo))
        o_v[pl.ds(i * _L, _L)] = jnp.where(pick_hi, vhi, vlo)

    pltpu.sync_copy(o_v, out_hbm.at[pl.ds(base, _PER)])


@functools.cache
def _make_search():
    mesh = plsc.VectorSubcoreMesh(
        core_axis_name="c", subcore_axis_name="s", num_cores=_NC, num_subcores=_NS
    )
    return pl.kernel(
        _body,
        out_type=jax.ShapeDtypeStruct((_N,), jnp.float32),
        mesh=mesh,
        scratch_types=[
            pltpu.VMEM((_PER,), jnp.float32),
            pltpu.VMEM((_PER,), jnp.float32),
            pltpu.VMEM((_K,), jnp.float32),
            pltpu.VMEM((_K,), jnp.float32),
            pltpu.VMEM((_K,), jnp.float32),
            pltpu.VMEM((_EPS,), jnp.float32),
            pltpu.VMEM((_EPS,), jnp.int32),
            pltpu.VMEM((_EPS,), jnp.float32),
            pltpu.VMEM((_K,), jnp.float32),
            pltpu.VMEM((_K,), jnp.int32),
            pltpu.VMEM((_K,), jnp.float32),
            pltpu.VMEM_SHARED((_K,), jnp.float32),
            pltpu.VMEM_SHARED((_K,), jnp.int32),
            pltpu.VMEM_SHARED((_K,), jnp.float32),
        ],
        compiler_params=pltpu.CompilerParams(needs_layout_passes=False),
    )


def kernel(h, embeddings):
    q = _make_search()(h.reshape(_N), embeddings.reshape(_K))
    return q.reshape(h.shape)


# speedup vs baseline: 1.1503x; 1.1503x over previous
import functools

import jax
import jax.numpy as jnp
from jax import lax
from jax.experimental import pallas as pl
from jax.experimental.pallas import tpu as pltpu
from jax.experimental.pallas import tpu_sc as plsc

_K = 512
_N = 2 * 576 * 96
_NC = 2
_NS = 16
_NW = _NC * _NS
_PER = _N // _NW
_L = 16


def _prep_body(er_ref, ec_ref, sv_ref, mi_ref):
    a = jnp.broadcast_to(er_ref[...], (_K, _K))
    b = jnp.broadcast_to(ec_ref[...], (_K, _K))
    ii = lax.broadcasted_iota(jnp.int32, (_K, _K), 0)
    kk = lax.broadcasted_iota(jnp.int32, (_K, _K), 1)
    lt = (a < b).astype(jnp.int32)
    eq_before = ((a == b) & (kk < ii)).astype(jnp.int32)
    rank = jnp.sum(lt + eq_before, axis=1, keepdims=True)
    onehot = rank == kk
    sv = jnp.sum(jnp.where(onehot, b, 0.0), axis=0, keepdims=True)
    eqv = b == jnp.broadcast_to(sv, (_K, _K))
    mi = jnp.max(jnp.where(eqv, ii, -1), axis=0, keepdims=True)
    sv_ref[...] = sv
    mi_ref[...] = mi.astype(jnp.float32)


_prep = pl.pallas_call(
    _prep_body,
    out_shape=(
        jax.ShapeDtypeStruct((1, _K), jnp.float32),
        jax.ShapeDtypeStruct((1, _K), jnp.float32),
    ),
)

def _search_body(h_hbm, sv_hbm, mi_hbm, out_hbm, x_v, o_v, sv_v, mi_v):
    wid = lax.axis_index("s") * _NC + lax.axis_index("c")
    base = wid * _PER
    pltpu.sync_copy(sv_hbm, sv_v)
    pltpu.sync_copy(mi_hbm, mi_v)
    pltpu.sync_copy(h_hbm.at[pl.ds(base, _PER)], x_v)

    @plsc.parallel_loop(0, _PER // _L, unroll=24)
    def body(i):
        x = x_v[pl.ds(i * _L, _L)]
        j = jnp.zeros((_L,), jnp.int32)
        step = _K // 2
        while step >= 1:
            probe = j + (step - 1)
            v = plsc.load_gather(sv_v, [probe])
            j = jnp.where(v < x, j + step, j)
            step //= 2
        lo = jnp.maximum(j - 1, 0)
        vlo = plsc.load_gather(sv_v, [lo])
        vhi = plsc.load_gather(sv_v, [j])
        milo = plsc.load_gather(mi_v, [lo])
        mihi = plsc.load_gather(mi_v, [j])
        dlo = jnp.abs(x - vlo)
        dhi = jnp.abs(vhi - x)
        pick_hi = (dhi < dlo) | ((dhi == dlo) & (mihi > milo))
        o_v[pl.ds(i * _L, _L)] = jnp.where(pick_hi, vhi, vlo)

    pltpu.sync_copy(o_v, out_hbm.at[pl.ds(base, _PER)])


@functools.cache
def _make_search():
    mesh = plsc.VectorSubcoreMesh(
        core_axis_name="c", subcore_axis_name="s", num_cores=_NC, num_subcores=_NS
    )
    return pl.kernel(
        _search_body,
        out_type=jax.ShapeDtypeStruct((_N,), jnp.float32),
        mesh=mesh,
        scratch_types=[
            pltpu.VMEM((_PER,), jnp.float32),
            pltpu.VMEM((_PER,), jnp.float32),
            pltpu.VMEM((_K,), jnp.float32),
            pltpu.VMEM((_K,), jnp.float32),
        ],
        compiler_params=pltpu.CompilerParams(needs_layout_passes=False),
    )


def kernel(h, embeddings):
    sv, mi = _prep(embeddings.reshape(1, _K), embeddings.reshape(_K, 1))
    q = _make_search()(h.reshape(_N), sv.reshape(_K), mi.reshape(_K))
    return q.reshape(h.shape)


# speedup vs baseline: 1.2334x; 1.0722x over previous
import functools

import jax
import jax.numpy as jnp
from jax import lax
from jax.experimental import pallas as pl
from jax.experimental.pallas import tpu as pltpu
from jax.experimental.pallas import tpu_sc as plsc

_K = 512
_N = 2 * 576 * 96
_NC = 2
_NS = 16
_NW = _NC * _NS
_PER = _N // _NW
_L = 16
_R = _K * _L


def _prep_body(er_ref, ec_ref, svr_ref, mir_ref):
    a = jnp.broadcast_to(er_ref[...], (_K, _K))
    b = jnp.broadcast_to(ec_ref[...], (_K, _K))
    ii = lax.broadcasted_iota(jnp.int32, (_K, _K), 0)
    kk = lax.broadcasted_iota(jnp.int32, (_K, _K), 1)
    before = (b < a) | ((b == a) & (ii < kk))
    rank = jnp.sum(before.astype(jnp.int32), axis=0, keepdims=True)
    onehot = ii == jnp.broadcast_to(rank, (_K, _K))
    svcol = jnp.sum(jnp.where(onehot, a, 0.0), axis=1, keepdims=True)
    eqv = a == jnp.broadcast_to(svcol, (_K, _K))
    micol = jnp.max(jnp.where(eqv, kk, -1), axis=1, keepdims=True)
    svr_ref[...] = jnp.broadcast_to(svcol, (_K, _L))
    mir_ref[...] = jnp.broadcast_to(micol.astype(jnp.float32), (_K, _L))


_prep = pl.pallas_call(
    _prep_body,
    out_shape=(
        jax.ShapeDtypeStruct((_K, _L), jnp.float32),
        jax.ShapeDtypeStruct((_K, _L), jnp.float32),
    ),
)


def _search_body(h_hbm, svr_hbm, mir_hbm, out_hbm, x_v, o_v, svr_v, mir_v):
    wid = lax.axis_index("s") * _NC + lax.axis_index("c")
    base = wid * _PER
    pltpu.sync_copy(svr_hbm, svr_v)
    pltpu.sync_copy(mir_hbm, mir_v)
    pltpu.sync_copy(h_hbm.at[pl.ds(base, _PER)], x_v)
    lane = lax.iota(jnp.int32, _L)

    @plsc.parallel_loop(0, _PER // _L, unroll=24)
    def body(i):
        x = x_v[pl.ds(i * _L, _L)]
        j16 = jnp.zeros((_L,), jnp.int32)
        step16 = (_K // 2) * _L
        while step16 >= _L:
            v = plsc.load_gather(svr_v, [j16 + (lane + (step16 - _L))])
            j16 = jnp.where(v < x, j16 + step16, j16)
            step16 //= 2
        lovec = jnp.maximum(j16 - _L, 0) + lane
        hivec = j16 + lane
        vlo = plsc.load_gather(svr_v, [lovec])
        vhi = plsc.load_gather(svr_v, [hivec])
        milo = plsc.load_gather(mir_v, [lovec])
        mihi = plsc.load_gather(mir_v, [hivec])
        dlo = jnp.abs(x - vlo)
        dhi = jnp.abs(vhi - x)
        pick_hi = (dhi < dlo) | ((dhi == dlo) & (mihi > milo))
        o_v[pl.ds(i * _L, _L)] = jnp.where(pick_hi, vhi, vlo)

    pltpu.sync_copy(o_v, out_hbm.at[pl.ds(base, _PER)])


@functools.cache
def _make_search():
    mesh = plsc.VectorSubcoreMesh(
        core_axis_name="c", subcore_axis_name="s", num_cores=_NC, num_subcores=_NS
    )
    return pl.kernel(
        _search_body,
        out_type=jax.ShapeDtypeStruct((_N,), jnp.float32),
        mesh=mesh,
        scratch_types=[
            pltpu.VMEM((_PER,), jnp.float32),
            pltpu.VMEM((_PER,), jnp.float32),
            pltpu.VMEM((_R,), jnp.float32),
            pltpu.VMEM((_R,), jnp.float32),
        ],
        compiler_params=pltpu.CompilerParams(needs_layout_passes=False),
    )


def kernel(h, embeddings):
    svr, mir = _prep(embeddings.reshape(1, _K), embeddings.reshape(_K, 1))
    q = _make_search()(h.reshape(_N), svr.reshape(_R), mir.reshape(_R))
    return q.reshape(h.shape)
